# baseline (speedup 1.0000x reference)
import functools

import jax
import jax.numpy as jnp
from jax import lax
from jax.experimental import pallas as pl
from jax.experimental.pallas import tpu as pltpu
from jax.experimental.pallas import tpu_sc as plsc

NUM_EXPERTS = 8
N_EMBD = 1024
EXPERT_DIM = 128
VOCAB = 100000
N_TOKENS = 4096

NC, NS, L = 2, 16, 16
NW = NC * NS
CHUNK = N_TOKENS // NW
VPW = CHUNK // L
SUB = 32
NSUB = CHUNK // SUB

BN_SH = 1024
BN_RT = 512
NB_RT = N_TOKENS // BN_RT
NPAIR = NB_RT + NUM_EXPERTS - 1


def _sc_mesh():
    return plsc.VectorSubcoreMesh(core_axis_name="c", subcore_axis_name="s",
                                  num_cores=NC, num_subcores=NS)


def _take(v, idx):
    return lax.gather(
        v, idx[:, None],
        dimension_numbers=lax.GatherDimensionNumbers(
            offset_dims=(), collapsed_slice_dims=(0,), start_index_map=(0,)),
        slice_sizes=(1,),
        mode=lax.GatherScatterMode.PROMISE_IN_BOUNDS)


_LANES = None


def _vhelpers():
    lanes = lax.iota(jnp.int32, L)
    ones = jnp.ones((L,), jnp.int32)
    zeros = jnp.zeros((L,), jnp.int32)
    last = jnp.full((L,), L - 1, jnp.int32)

    def b2i(m):
        return jnp.where(m, ones, zeros)

    def vcumsum(v):
        for s in (1, 2, 4, 8):
            sh = _take(v, jnp.maximum(lanes - s, 0))
            v = v + jnp.where(lanes >= s, sh, zeros)
        return v

    def vbroadcast_last(v):
        return _take(v, last)

    return lanes, ones, zeros, b2i, vcumsum, vbroadcast_last


def _expert_of_vec(v):
    v = jnp.minimum(jnp.maximum(v, 0), VOCAB - 1)
    return lax.rem(v, NUM_EXPERTS)


def _hist_body(tids_hbm, counts_hbm, tids_v, cnt_v):
    wid = lax.axis_index("s") * NC + lax.axis_index("c")
    base = wid * CHUNK
    lanes, ones, zeros, b2i, vcumsum, vlast = _vhelpers()
    pltpu.sync_copy(tids_hbm.at[pl.ds(base, CHUNK)], tids_v)
    cnt = zeros
    for j in range(VPW):
        e = _expert_of_vec(tids_v[pl.ds(j * L, L)])
        for ex in range(NUM_EXPERTS):
            pc = vlast(vcumsum(b2i(e == ex)))
            cnt = cnt + jnp.where(lanes == ex, pc, zeros)
    cnt_v[...] = cnt
    pltpu.sync_copy(cnt_v, counts_hbm.at[wid])


def _make_hist():
    return functools.partial(
        pl.kernel,
        out_type=jax.ShapeDtypeStruct((NW, L), jnp.int32),
        mesh=_sc_mesh(),
        scratch_types=[
            pltpu.VMEM((CHUNK,), jnp.int32),
            pltpu.VMEM((L,), jnp.int32),
        ],
    )(_hist_body)


def _dispatch_body(tids_hbm, x_hbm, counts_hbm, pos_hbm, seg_hbm, xs_hbm,
                   tids_v, cnts_v, seg_v, idx0, idx1, idx2,
                   idx3, xbuf, sem):
    wid = lax.axis_index("s") * NC + lax.axis_index("c")
    base = wid * CHUNK
    idxrefs = (idx0, idx1, idx2, idx3)
    lanes, ones, zeros, b2i, vcumsum, vlast = _vhelpers()

    pltpu.sync_copy(tids_hbm.at[pl.ds(base, CHUNK)], tids_v)
    pltpu.sync_copy(counts_hbm, cnts_v)

    tot = zeros
    pre = zeros
    for w in range(NW):
        row = cnts_v[w]
        tot = tot + row
        pre = pre + jnp.where(w < wid, row, zeros)

    seg_vec = vcumsum(tot) - tot
    seg_v[...] = seg_vec

    @pl.when(wid == 0)
    def _():
        pltpu.sync_copy(seg_v, seg_hbm)

    run = seg_vec + pre
    for j in range(VPW):
        e = _expert_of_vec(tids_v[pl.ds(j * L, L)])
        posv = zeros
        for ex in range(NUM_EXPERTS):
            m = e == ex
            c = vcumsum(b2i(m))
            runex = _take(run, jnp.full((L,), ex, jnp.int32))
            posv = jnp.where(m, runex + c - 1, posv)
            run = run + jnp.where(lanes == ex, vlast(c), zeros)
        idxrefs[j // 2][pl.ds((j % 2) * L, L)] = posv

    for k in range(NSUB):
        pltpu.sync_copy(idxrefs[k], pos_hbm.at[pl.ds(base + k * SUB, SUB)])

    for k in range(NSUB):
        pltpu.sync_copy(x_hbm.at[pl.ds(base + k * SUB, SUB)], xbuf)
        pltpu.async_copy(xbuf, xs_hbm.at[idxrefs[k]], sem).wait()


def _make_dispatch():
    return functools.partial(
        pl.kernel,
        out_type=[
            jax.ShapeDtypeStruct((N_TOKENS,), jnp.int32),
            jax.ShapeDtypeStruct((L,), jnp.int32),
            jax.ShapeDtypeStruct((N_TOKENS, N_EMBD), jnp.float32),
        ],
        mesh=_sc_mesh(),
        scratch_types=[
            pltpu.VMEM((CHUNK,), jnp.int32),
            pltpu.VMEM((NW, L), jnp.int32),
            pltpu.VMEM((L,), jnp.int32),
            pltpu.VMEM((SUB,), jnp.int32),
            pltpu.VMEM((SUB,), jnp.int32),
            pltpu.VMEM((SUB,), jnp.int32),
            pltpu.VMEM((SUB,), jnp.int32),
            pltpu.VMEM((SUB, N_EMBD), jnp.float32),
            pltpu.SemaphoreType.DMA,
        ],
    )(_dispatch_body)


def _combine_body(sh_hbm, rs_hbm, pos_hbm, out_hbm, idxbuf, gbuf, sbuf, sem):
    wid = lax.axis_index("s") * NC + lax.axis_index("c")
    base = wid * CHUNK

    for k in range(NSUB):
        lo = base + k * SUB
        pltpu.sync_copy(pos_hbm.at[pl.ds(lo, SUB)], idxbuf)
        cp = pltpu.async_copy(rs_hbm.at[idxbuf], gbuf, sem)
        pltpu.sync_copy(sh_hbm.at[pl.ds(lo, SUB)], sbuf)
        cp.wait()

        def addrow(r, _):
            for c in range(N_EMBD // L):
                gbuf[r, pl.ds(c * L, L)] = (
                    gbuf[r, pl.ds(c * L, L)] + sbuf[r, pl.ds(c * L, L)])
            return 0

        lax.fori_loop(0, SUB, addrow, 0)
        pltpu.sync_copy(gbuf, out_hbm.at[pl.ds(lo, SUB)])


def _make_combine():
    return functools.partial(
        pl.kernel,
        out_type=jax.ShapeDtypeStruct((N_TOKENS, N_EMBD), jnp.float32),
        mesh=_sc_mesh(),
        scratch_types=[
            pltpu.VMEM((SUB,), jnp.int32),
            pltpu.VMEM((SUB, N_EMBD), jnp.float32),
            pltpu.VMEM((SUB, N_EMBD), jnp.float32),
            pltpu.SemaphoreType.DMA,
        ],
    )(_combine_body)


def _shared_body(x_ref, wsg_ref, wsu_ref, wsd_ref, out_ref):
    x = x_ref[...]
    g = jnp.dot(x, wsg_ref[...], preferred_element_type=jnp.float32)
    u = jnp.dot(x, wsu_ref[...], preferred_element_type=jnp.float32)
    s = g * jax.nn.sigmoid(g) * u
    out_ref[...] = jnp.dot(s, wsd_ref[...], preferred_element_type=jnp.float32)


def _shared_call(flat_x, wsg, wsu, wsd):
    h = N_EMBD
    return pl.pallas_call(
        _shared_body,
        grid=(N_TOKENS // BN_SH,),
        in_specs=[
            pl.BlockSpec((BN_SH, h), lambda i: (i, 0)),
            pl.BlockSpec((h, h), lambda i: (0, 0)),
            pl.BlockSpec((h, h), lambda i: (0, 0)),
            pl.BlockSpec((h, h), lambda i: (0, 0)),
        ],
        out_specs=pl.BlockSpec((BN_SH, h), lambda i: (i, 0)),
        out_shape=jax.ShapeDtypeStruct((N_TOKENS, h), jnp.float32),
    )(flat_x, wsg, wsu, wsd)


def _routed_body(meta_ref, xs_ref, wga_ref, wua_ref, wda_ref, out_ref):
    k = pl.program_id(0)
    e = meta_ref[1, k]
    lo = meta_ref[2, k]
    hi = meta_ref[3, k]
    first = meta_ref[4, k]
    off = pl.multiple_of(e * EXPERT_DIM, EXPERT_DIM)
    x = xs_ref[...]
    g = jnp.dot(x, wga_ref[:, pl.ds(off, EXPERT_DIM)],
                preferred_element_type=jnp.float32)
    u = jnp.dot(x, wua_ref[:, pl.ds(off, EXPERT_DIM)],
                preferred_element_type=jnp.float32)
    rows = lax.broadcasted_iota(jnp.int32, (BN_RT, EXPERT_DIM), 0)
    m = ((rows >= lo) & (rows < hi)).astype(jnp.float32)
    inter = g * jax.nn.sigmoid(g) * u * m
    r = jnp.dot(inter, wda_ref[pl.ds(off, EXPERT_DIM), :],
                preferred_element_type=jnp.float32)

    @pl.when(first == 1)
    def _():
        out_ref[...] = r

    @pl.when(first == 0)
    def _():
        out_ref[...] = out_ref[...] + r


def _routed_call(meta, xs, wga, wua, wda):
    h = N_EMBD
    grid_spec = pltpu.PrefetchScalarGridSpec(
        num_scalar_prefetch=1,
        grid=(NPAIR,),
        in_specs=[
            pl.BlockSpec((BN_RT, h), lambda k, meta: (meta[0, k], 0)),
            pl.BlockSpec((h, NUM_EXPERTS * EXPERT_DIM), lambda k, meta: (0, 0)),
            pl.BlockSpec((h, NUM_EXPERTS * EXPERT_DIM), lambda k, meta: (0, 0)),
            pl.BlockSpec((NUM_EXPERTS * EXPERT_DIM, h), lambda k, meta: (0, 0)),
        ],
        out_specs=pl.BlockSpec((BN_RT, h), lambda k, meta: (meta[0, k], 0)),
    )
    return pl.pallas_call(
        _routed_body,
        grid_spec=grid_spec,
        out_shape=jax.ShapeDtypeStruct((N_TOKENS, h), jnp.float32),
    )(meta, xs, wga, wua, wda)


def _pair_meta(seg16):
    seg = seg16[:NUM_EXPERTS + 1]
    block_starts = jnp.arange(NB_RT, dtype=jnp.int32) * BN_RT
    bps = jnp.sort(jnp.concatenate([block_starts, seg[1:NUM_EXPERTS]]))
    ends = jnp.concatenate(
        [bps[1:], jnp.array([N_TOKENS], dtype=jnp.int32)])
    blk = jnp.minimum(bps // BN_RT, NB_RT - 1)
    e_of = jnp.clip(
        jnp.searchsorted(seg, bps, side="right").astype(jnp.int32) - 1,
        0, NUM_EXPERTS - 1)
    lo_rel = bps - blk * BN_RT
    hi_rel = ends - blk * BN_RT
    first = jnp.concatenate(
        [jnp.ones((1,), jnp.int32),
         (blk[1:] != blk[:-1]).astype(jnp.int32)])
    return jnp.stack([blk, e_of, lo_rel, hi_rel, first]).astype(jnp.int32)


def kernel(x, token_ids, gate_proj_w, up_proj_w, down_proj_w, shared_gate_w,
           shared_up_w, shared_down_w):
    b, t, h = x.shape
    flat_x = x.reshape(N_TOKENS, h)
    tids = token_ids.reshape(N_TOKENS).astype(jnp.int32)
    wga = jnp.transpose(gate_proj_w, (1, 0, 2)).reshape(h, NUM_EXPERTS * EXPERT_DIM)
    wua = jnp.transpose(up_proj_w, (1, 0, 2)).reshape(h, NUM_EXPERTS * EXPERT_DIM)
    wda = down_proj_w.reshape(NUM_EXPERTS * EXPERT_DIM, h)

    counts = _make_hist()(tids)
    pos, seg16, xs = _make_dispatch()(tids, flat_x, counts)
    shared_out = _shared_call(flat_x, shared_gate_w, shared_up_w, shared_down_w)
    meta = _pair_meta(seg16)
    routed_sorted = _routed_call(meta, xs, wga, wua, wda)
    out = _make_combine()(shared_out, routed_sorted, pos)
    return out.reshape(b, t, h)


# speedup vs baseline: 1.0667x; 1.0667x over previous
import functools

import jax
import jax.numpy as jnp
from jax import lax
from jax.experimental import pallas as pl
from jax.experimental.pallas import tpu as pltpu
from jax.experimental.pallas import tpu_sc as plsc

NUM_EXPERTS = 8
N_EMBD = 1024
EXPERT_DIM = 128
VOCAB = 100000
N_TOKENS = 4096

NC, NS, L = 2, 16, 16
NW = NC * NS
CHUNK = N_TOKENS // NW
VPW = CHUNK // L
SUB = 32
NSUB = CHUNK // SUB

BN_SH = 1024
BN_RT = 512
NB_RT = N_TOKENS // BN_RT
NPAIR = NB_RT + NUM_EXPERTS - 1


def _sc_mesh():
    return plsc.VectorSubcoreMesh(core_axis_name="c", subcore_axis_name="s",
                                  num_cores=NC, num_subcores=NS)


def _take(v, idx):
    return lax.gather(
        v, idx[:, None],
        dimension_numbers=lax.GatherDimensionNumbers(
            offset_dims=(), collapsed_slice_dims=(0,), start_index_map=(0,)),
        slice_sizes=(1,),
        mode=lax.GatherScatterMode.PROMISE_IN_BOUNDS)


_LANES = None


def _vhelpers():
    lanes = lax.iota(jnp.int32, L)
    ones = jnp.ones((L,), jnp.int32)
    zeros = jnp.zeros((L,), jnp.int32)
    last = jnp.full((L,), L - 1, jnp.int32)

    def b2i(m):
        return jnp.where(m, ones, zeros)

    def vcumsum(v):
        for s in (1, 2, 4, 8):
            sh = _take(v, jnp.maximum(lanes - s, 0))
            v = v + jnp.where(lanes >= s, sh, zeros)
        return v

    def vbroadcast_last(v):
        return _take(v, last)

    return lanes, ones, zeros, b2i, vcumsum, vbroadcast_last


def _expert_of_vec(v):
    v = jnp.minimum(jnp.maximum(v, 0), VOCAB - 1)
    return lax.rem(v, NUM_EXPERTS)


def _hist_body(tids_hbm, counts_hbm, tids_v, cnt_v):
    wid = lax.axis_index("s") * NC + lax.axis_index("c")
    base = wid * CHUNK
    lanes, ones, zeros, b2i, vcumsum, vlast = _vhelpers()
    pltpu.sync_copy(tids_hbm.at[pl.ds(base, CHUNK)], tids_v)
    cnt = zeros
    for j in range(VPW):
        e = _expert_of_vec(tids_v[pl.ds(j * L, L)])
        for ex in range(NUM_EXPERTS):
            pc = vlast(vcumsum(b2i(e == ex)))
            cnt = cnt + jnp.where(lanes == ex, pc, zeros)
    cnt_v[...] = cnt
    pltpu.sync_copy(cnt_v, counts_hbm.at[wid])


def _make_hist():
    return functools.partial(
        pl.kernel,
        out_type=jax.ShapeDtypeStruct((NW, L), jnp.int32),
        mesh=_sc_mesh(),
        scratch_types=[
            pltpu.VMEM((CHUNK,), jnp.int32),
            pltpu.VMEM((L,), jnp.int32),
        ],
    )(_hist_body)


def _dispatch_body(tids_hbm, x_hbm, counts_hbm, pos_hbm, seg_hbm, xs_hbm,
                   tids_v, cnts_v, seg_v, idx0, idx1, idx2,
                   idx3, xbuf, sem):
    wid = lax.axis_index("s") * NC + lax.axis_index("c")
    base = wid * CHUNK
    idxrefs = (idx0, idx1, idx2, idx3)
    lanes, ones, zeros, b2i, vcumsum, vlast = _vhelpers()

    pltpu.sync_copy(tids_hbm.at[pl.ds(base, CHUNK)], tids_v)
    pltpu.sync_copy(counts_hbm, cnts_v)

    tot = zeros
    pre = zeros
    for w in range(NW):
        row = cnts_v[w]
        tot = tot + row
        pre = pre + jnp.where(w < wid, row, zeros)

    seg_vec = vcumsum(tot) - tot
    seg_v[...] = seg_vec

    @pl.when(wid == 0)
    def _():
        pltpu.sync_copy(seg_v, seg_hbm)

    run = seg_vec + pre
    for j in range(VPW):
        e = _expert_of_vec(tids_v[pl.ds(j * L, L)])
        posv = zeros
        for ex in range(NUM_EXPERTS):
            m = e == ex
            c = vcumsum(b2i(m))
            runex = _take(run, jnp.full((L,), ex, jnp.int32))
            posv = jnp.where(m, runex + c - 1, posv)
            run = run + jnp.where(lanes == ex, vlast(c), zeros)
        idxrefs[j // 2][pl.ds((j % 2) * L, L)] = posv

    for k in range(NSUB):
        pltpu.sync_copy(idxrefs[k], pos_hbm.at[pl.ds(base + k * SUB, SUB)])

    for k in range(NSUB):
        pltpu.sync_copy(x_hbm.at[pl.ds(base + k * SUB, SUB)], xbuf)
        pltpu.async_copy(xbuf, xs_hbm.at[idxrefs[k]], sem).wait()


def _make_dispatch():
    return functools.partial(
        pl.kernel,
        out_type=[
            jax.ShapeDtypeStruct((N_TOKENS,), jnp.int32),
            jax.ShapeDtypeStruct((L,), jnp.int32),
            jax.ShapeDtypeStruct((N_TOKENS, N_EMBD), jnp.float32),
        ],
        mesh=_sc_mesh(),
        scratch_types=[
            pltpu.VMEM((CHUNK,), jnp.int32),
            pltpu.VMEM((NW, L), jnp.int32),
            pltpu.VMEM((L,), jnp.int32),
            pltpu.VMEM((SUB,), jnp.int32),
            pltpu.VMEM((SUB,), jnp.int32),
            pltpu.VMEM((SUB,), jnp.int32),
            pltpu.VMEM((SUB,), jnp.int32),
            pltpu.VMEM((SUB, N_EMBD), jnp.float32),
            pltpu.SemaphoreType.DMA,
        ],
    )(_dispatch_body)


def _unsort_body(is_hbm, pos_hbm, out_hbm, idxbuf, gbuf, sem):
    wid = lax.axis_index("s") * NC + lax.axis_index("c")
    base = wid * CHUNK
    pltpu.sync_copy(pos_hbm.at[pl.ds(base, CHUNK)], idxbuf)
    pltpu.async_copy(is_hbm.at[idxbuf], gbuf, sem).wait()
    pltpu.sync_copy(gbuf, out_hbm.at[pl.ds(base, CHUNK)])


def _make_unsort():
    return functools.partial(
        pl.kernel,
        out_type=jax.ShapeDtypeStruct((N_TOKENS, EXPERT_DIM), jnp.float32),
        mesh=_sc_mesh(),
        scratch_types=[
            pltpu.VMEM((CHUNK,), jnp.int32),
            pltpu.VMEM((CHUNK, EXPERT_DIM), jnp.float32),
            pltpu.SemaphoreType.DMA,
        ],
    )(_unsort_body)


def _sgu_body(x_ref, wsg_ref, wsu_ref, out_ref):
    x = x_ref[...]
    g = jnp.dot(x, wsg_ref[...], preferred_element_type=jnp.float32)
    u = jnp.dot(x, wsu_ref[...], preferred_element_type=jnp.float32)
    out_ref[...] = g * jax.nn.sigmoid(g) * u


def _sgu_call(flat_x, wsg, wsu):
    h = N_EMBD
    return pl.pallas_call(
        _sgu_body,
        grid=(N_TOKENS // BN_SH,),
        in_specs=[
            pl.BlockSpec((BN_SH, h), lambda i: (i, 0)),
            pl.BlockSpec((h, h), lambda i: (0, 0)),
            pl.BlockSpec((h, h), lambda i: (0, 0)),
        ],
        out_specs=pl.BlockSpec((BN_SH, h), lambda i: (i, 0)),
        out_shape=jax.ShapeDtypeStruct((N_TOKENS, h), jnp.float32),
    )(flat_x, wsg, wsu)


def _final_body(tid_ref, s_ref, inter_ref, wsd_ref, wda_ref, out_ref):
    s = s_ref[...]
    acc = jnp.dot(s, wsd_ref[...], preferred_element_type=jnp.float32)
    eids = lax.rem(tid_ref[0, 0, :], NUM_EXPERTS).reshape(BN_SH, 1)
    col_expert = lax.broadcasted_iota(
        jnp.int32, (BN_SH, NUM_EXPERTS * EXPERT_DIM), 1) // EXPERT_DIM
    mask = (col_expert == eids).astype(jnp.float32)
    inter = inter_ref[...]
    wide = jnp.concatenate([inter] * NUM_EXPERTS, axis=1) * mask
    out_ref[...] = acc + jnp.dot(wide, wda_ref[...],
                                 preferred_element_type=jnp.float32)


def _final_call(tid, s, inter_un, wsd, wda):
    h = N_EMBD
    nb = N_TOKENS // BN_SH
    return pl.pallas_call(
        _final_body,
        grid=(nb,),
        in_specs=[
            pl.BlockSpec((1, 1, BN_SH), lambda i: (i, 0, 0)),
            pl.BlockSpec((BN_SH, h), lambda i: (i, 0)),
            pl.BlockSpec((BN_SH, EXPERT_DIM), lambda i: (i, 0)),
            pl.BlockSpec((h, h), lambda i: (0, 0)),
            pl.BlockSpec((h, h), lambda i: (0, 0)),
        ],
        out_specs=pl.BlockSpec((BN_SH, h), lambda i: (i, 0)),
        out_shape=jax.ShapeDtypeStruct((N_TOKENS, h), jnp.float32),
    )(tid, s, inter_un, wsd, wda)


def _routed_body(meta_ref, xs_ref, wga_ref, wua_ref, out_ref):
    k = pl.program_id(0)
    e = meta_ref[1, k]
    lo = meta_ref[2, k]
    hi = meta_ref[3, k]
    first = meta_ref[4, k]
    off = pl.multiple_of(e * EXPERT_DIM, EXPERT_DIM)
    x = xs_ref[...]
    g = jnp.dot(x, wga_ref[:, pl.ds(off, EXPERT_DIM)],
                preferred_element_type=jnp.float32)
    u = jnp.dot(x, wua_ref[:, pl.ds(off, EXPERT_DIM)],
                preferred_element_type=jnp.float32)
    rows = lax.broadcasted_iota(jnp.int32, (BN_RT, EXPERT_DIM), 0)
    m = ((rows >= lo) & (rows < hi)).astype(jnp.float32)
    r = g * jax.nn.sigmoid(g) * u * m

    @pl.when(first == 1)
    def _():
        out_ref[...] = r

    @pl.when(first == 0)
    def _():
        out_ref[...] = out_ref[...] + r


def _routed_call(meta, xs, wga, wua):
    h = N_EMBD
    grid_spec = pltpu.PrefetchScalarGridSpec(
        num_scalar_prefetch=1,
        grid=(NPAIR,),
        in_specs=[
            pl.BlockSpec((BN_RT, h), lambda k, meta: (meta[0, k], 0)),
            pl.BlockSpec((h, NUM_EXPERTS * EXPERT_DIM), lambda k, meta: (0, 0)),
            pl.BlockSpec((h, NUM_EXPERTS * EXPERT_DIM), lambda k, meta: (0, 0)),
        ],
        out_specs=pl.BlockSpec((BN_RT, EXPERT_DIM), lambda k, meta: (meta[0, k], 0)),
    )
    return pl.pallas_call(
        _routed_body,
        grid_spec=grid_spec,
        out_shape=jax.ShapeDtypeStruct((N_TOKENS, EXPERT_DIM), jnp.float32),
    )(meta, xs, wga, wua)


def _pair_meta(seg16):
    seg = seg16[:NUM_EXPERTS + 1]
    block_starts = jnp.arange(NB_RT, dtype=jnp.int32) * BN_RT
    bps = jnp.sort(jnp.concatenate([block_starts, seg[1:NUM_EXPERTS]]))
    ends = jnp.concatenate(
        [bps[1:], jnp.array([N_TOKENS], dtype=jnp.int32)])
    blk = jnp.minimum(bps // BN_RT, NB_RT - 1)
    e_of = jnp.clip(
        jnp.searchsorted(seg, bps, side="right").astype(jnp.int32) - 1,
        0, NUM_EXPERTS - 1)
    lo_rel = bps - blk * BN_RT
    hi_rel = ends - blk * BN_RT
    first = jnp.concatenate(
        [jnp.ones((1,), jnp.int32),
         (blk[1:] != blk[:-1]).astype(jnp.int32)])
    return jnp.stack([blk, e_of, lo_rel, hi_rel, first]).astype(jnp.int32)


def kernel(x, token_ids, gate_proj_w, up_proj_w, down_proj_w, shared_gate_w,
           shared_up_w, shared_down_w):
    b, t, h = x.shape
    flat_x = x.reshape(N_TOKENS, h)
    tids = token_ids.reshape(N_TOKENS).astype(jnp.int32)
    wga = jnp.transpose(gate_proj_w, (1, 0, 2)).reshape(h, NUM_EXPERTS * EXPERT_DIM)
    wua = jnp.transpose(up_proj_w, (1, 0, 2)).reshape(h, NUM_EXPERTS * EXPERT_DIM)
    wda = down_proj_w.reshape(NUM_EXPERTS * EXPERT_DIM, h)

    tid3 = token_ids.reshape(N_TOKENS // BN_SH, 1, BN_SH).astype(jnp.int32)

    counts = _make_hist()(tids)
    pos, seg16, xs = _make_dispatch()(tids, flat_x, counts)
    s = _sgu_call(flat_x, shared_gate_w, shared_up_w)
    meta = _pair_meta(seg16)
    inter_sorted = _routed_call(meta, xs, wga, wua)
    inter_un = _make_unsort()(inter_sorted, pos)
    out = _final_call(tid3, s, inter_un, shared_down_w, wda)
    return out.reshape(b, t, h)


# speedup vs baseline: 1.1744x; 1.1010x over previous
import functools

import jax
import jax.numpy as jnp
from jax import lax
from jax.experimental import pallas as pl
from jax.experimental.pallas import tpu as pltpu
from jax.experimental.pallas import tpu_sc as plsc

NUM_EXPERTS = 8
N_EMBD = 1024
EXPERT_DIM = 128
VOCAB = 100000
N_TOKENS = 4096

NC, NS, L = 2, 16, 16
NW = NC * NS
CHUNK = N_TOKENS // NW
VPW = CHUNK // L
SUB = 32
NSUB = CHUNK // SUB

BN_SH = 1024
BN_RT = 512
_BN_RT_SHIFT = 9
NB_RT = N_TOKENS // BN_RT
NPAIR = NB_RT + NUM_EXPERTS - 1


def _sc_mesh():
    return plsc.VectorSubcoreMesh(core_axis_name="c", subcore_axis_name="s",
                                  num_cores=NC, num_subcores=NS)


def _take(v, idx):
    return lax.gather(
        v, idx[:, None],
        dimension_numbers=lax.GatherDimensionNumbers(
            offset_dims=(), collapsed_slice_dims=(0,), start_index_map=(0,)),
        slice_sizes=(1,),
        mode=lax.GatherScatterMode.PROMISE_IN_BOUNDS)


_LANES = None


def _vhelpers():
    lanes = lax.iota(jnp.int32, L)
    ones = jnp.ones((L,), jnp.int32)
    zeros = jnp.zeros((L,), jnp.int32)
    last = jnp.full((L,), L - 1, jnp.int32)

    def b2i(m):
        return jnp.where(m, ones, zeros)

    def vcumsum(v):
        for s in (1, 2, 4, 8):
            sh = _take(v, jnp.maximum(lanes - s, 0))
            v = v + jnp.where(lanes >= s, sh, zeros)
        return v

    def vbroadcast_last(v):
        return _take(v, last)

    return lanes, ones, zeros, b2i, vcumsum, vbroadcast_last


def _expert_of_vec(v):
    v = jnp.minimum(jnp.maximum(v, 0), VOCAB - 1)
    return lax.rem(v, NUM_EXPERTS)


def _hist_body(tids_hbm, counts_hbm, tids_v, cnt_v):
    wid = lax.axis_index("s") * NC + lax.axis_index("c")
    base = wid * CHUNK
    lanes, ones, zeros, b2i, vcumsum, vlast = _vhelpers()
    pltpu.sync_copy(tids_hbm.at[pl.ds(base, CHUNK)], tids_v)
    cnt = zeros
    for j in range(VPW):
        e = _expert_of_vec(tids_v[pl.ds(j * L, L)])
        for ex in range(NUM_EXPERTS):
            pc = vlast(vcumsum(b2i(e == ex)))
            cnt = cnt + jnp.where(lanes == ex, pc, zeros)
    cnt_v[...] = cnt
    pltpu.sync_copy(cnt_v, counts_hbm.at[wid])


def _make_hist():
    return functools.partial(
        pl.kernel,
        out_type=jax.ShapeDtypeStruct((NW, L), jnp.int32),
        mesh=_sc_mesh(),
        scratch_types=[
            pltpu.VMEM((CHUNK,), jnp.int32),
            pltpu.VMEM((L,), jnp.int32),
        ],
    )(_hist_body)


def _rotate(v, lanes, sh):
    return _take(v, (lanes + sh) & (L - 1))


def _vsort16(v, lanes, zeros, ones):
    rank = zeros
    for sh in range(1, L):
        w = _rotate(v, lanes, sh)
        lt = jnp.where(w < v, ones, zeros)
        tie = jnp.where(w == v, jnp.where(lanes >= L - sh, ones, zeros), zeros)
        rank = rank + lt + tie
    out = zeros
    for sh in range(L):
        w = _rotate(v, lanes, sh) if sh else v
        r = _rotate(rank, lanes, sh) if sh else rank
        out = out + jnp.where(r == lanes, w, zeros)
    return out


def _dispatch_body(tids_hbm, x_hbm, counts_hbm, pos_hbm, meta_hbm, xs_hbm,
                   tids_v, cnts_v, meta_v, idx0, idx1, idx2,
                   idx3, xbuf0, xbuf1, sem):
    wid = lax.axis_index("s") * NC + lax.axis_index("c")
    base = wid * CHUNK
    idxrefs = (idx0, idx1, idx2, idx3)
    lanes, ones, zeros, b2i, vcumsum, vlast = _vhelpers()

    pltpu.sync_copy(tids_hbm.at[pl.ds(base, CHUNK)], tids_v)
    pltpu.sync_copy(counts_hbm, cnts_v)

    tot = zeros
    pre = zeros
    for w in range(NW):
        row = cnts_v[w]
        tot = tot + row
        pre = pre + jnp.where(w < wid, row, zeros)

    seg_vec = vcumsum(tot) - tot

    bp = jnp.where(lanes < NB_RT, lanes * BN_RT,
                   _take(seg_vec, jnp.maximum(lanes - (NB_RT - 1), 0)))
    bp = jnp.where(lanes == L - 1, jnp.full((L,), N_TOKENS, jnp.int32), bp)
    bp = _vsort16(bp, lanes, zeros, ones)
    ends = jnp.where(lanes == L - 1, jnp.full((L,), N_TOKENS, jnp.int32),
                     _rotate(bp, lanes, 1))
    blk = jnp.minimum(
        lax.shift_right_logical(bp, jnp.full((L,), _BN_RT_SHIFT, jnp.int32)),
        jnp.full((L,), NB_RT - 1, jnp.int32))
    e_of = zeros - 1
    for e in range(NUM_EXPERTS):
        sege = _take(seg_vec, jnp.full((L,), e, jnp.int32))
        e_of = e_of + jnp.where(sege <= bp, ones, zeros)
    e_of = jnp.minimum(jnp.maximum(e_of, 0),
                       jnp.full((L,), NUM_EXPERTS - 1, jnp.int32))
    blk_lo = blk * BN_RT
    first = jnp.where(lanes == 0, ones,
                      jnp.where(blk != _rotate(blk, lanes, L - 1),
                                ones, zeros))

    @pl.when(wid == 0)
    def _():
        meta_v[0, pl.ds(0, L)] = blk
        meta_v[1, pl.ds(0, L)] = e_of
        meta_v[2, pl.ds(0, L)] = bp - blk_lo
        meta_v[3, pl.ds(0, L)] = ends - blk_lo
        meta_v[4, pl.ds(0, L)] = first
        pltpu.sync_copy(meta_v, meta_hbm)

    run = seg_vec + pre
    for j in range(VPW):
        e = _expert_of_vec(tids_v[pl.ds(j * L, L)])
        posv = zeros
        for ex in range(NUM_EXPERTS):
            m = e == ex
            c = vcumsum(b2i(m))
            runex = _take(run, jnp.full((L,), ex, jnp.int32))
            posv = jnp.where(m, runex + c - 1, posv)
            run = run + jnp.where(lanes == ex, vlast(c), zeros)
        idxrefs[j // 2][pl.ds((j % 2) * L, L)] = posv

    for k in range(NSUB):
        pltpu.sync_copy(idxrefs[k], pos_hbm.at[pl.ds(base + k * SUB, SUB)])

    xbufs = (xbuf0, xbuf1)
    pltpu.sync_copy(x_hbm.at[pl.ds(base, SUB)], xbuf0)
    cps = [None, None]
    for k in range(NSUB):
        cps[k % 2] = pltpu.async_copy(xbufs[k % 2], xs_hbm.at[idxrefs[k]], sem)
        if k + 1 < NSUB:
            if cps[(k + 1) % 2] is not None:
                cps[(k + 1) % 2].wait()
            pltpu.sync_copy(x_hbm.at[pl.ds(base + (k + 1) * SUB, SUB)],
                            xbufs[(k + 1) % 2])
    cps[0].wait()
    cps[1].wait()


def _make_dispatch():
    return functools.partial(
        pl.kernel,
        out_type=[
            jax.ShapeDtypeStruct((N_TOKENS,), jnp.int32),
            jax.ShapeDtypeStruct((5, L), jnp.int32),
            jax.ShapeDtypeStruct((N_TOKENS, N_EMBD), jnp.float32),
        ],
        mesh=_sc_mesh(),
        scratch_types=[
            pltpu.VMEM((CHUNK,), jnp.int32),
            pltpu.VMEM((NW, L), jnp.int32),
            pltpu.VMEM((5, L), jnp.int32),
            pltpu.VMEM((SUB,), jnp.int32),
            pltpu.VMEM((SUB,), jnp.int32),
            pltpu.VMEM((SUB,), jnp.int32),
            pltpu.VMEM((SUB,), jnp.int32),
            pltpu.VMEM((SUB, N_EMBD), jnp.float32),
            pltpu.VMEM((SUB, N_EMBD), jnp.float32),
            pltpu.SemaphoreType.DMA,
        ],
    )(_dispatch_body)


def _unsort_body(is_hbm, pos_hbm, out_hbm, idxbuf, gbuf, sem):
    wid = lax.axis_index("s") * NC + lax.axis_index("c")
    base = wid * CHUNK
    pltpu.sync_copy(pos_hbm.at[pl.ds(base, CHUNK)], idxbuf)
    pltpu.async_copy(is_hbm.at[idxbuf], gbuf, sem).wait()
    pltpu.sync_copy(gbuf, out_hbm.at[pl.ds(base, CHUNK)])


def _make_unsort():
    return functools.partial(
        pl.kernel,
        out_type=jax.ShapeDtypeStruct((N_TOKENS, EXPERT_DIM), jnp.float32),
        mesh=_sc_mesh(),
        scratch_types=[
            pltpu.VMEM((CHUNK,), jnp.int32),
            pltpu.VMEM((CHUNK, EXPERT_DIM), jnp.float32),
            pltpu.SemaphoreType.DMA,
        ],
    )(_unsort_body)


def _sgu_body(x_ref, wsg_ref, wsu_ref, out_ref):
    x = x_ref[...]
    g = jnp.dot(x, wsg_ref[...], preferred_element_type=jnp.float32)
    u = jnp.dot(x, wsu_ref[...], preferred_element_type=jnp.float32)
    out_ref[...] = g * jax.nn.sigmoid(g) * u


def _sgu_call(flat_x, wsg, wsu):
    h = N_EMBD
    return pl.pallas_call(
        _sgu_body,
        grid=(N_TOKENS // BN_SH,),
        in_specs=[
            pl.BlockSpec((BN_SH, h), lambda i: (i, 0)),
            pl.BlockSpec((h, h), lambda i: (0, 0)),
            pl.BlockSpec((h, h), lambda i: (0, 0)),
        ],
        out_specs=pl.BlockSpec((BN_SH, h), lambda i: (i, 0)),
        out_shape=jax.ShapeDtypeStruct((N_TOKENS, h), jnp.float32),
    )(flat_x, wsg, wsu)


def _final_body(tid_ref, s_ref, inter_ref, wsd_ref, wda_ref, out_ref):
    s = s_ref[...]
    acc = jnp.dot(s, wsd_ref[...], preferred_element_type=jnp.float32)
    eids = lax.rem(tid_ref[0, 0, :], NUM_EXPERTS).reshape(BN_SH, 1)
    col_expert = lax.broadcasted_iota(
        jnp.int32, (BN_SH, NUM_EXPERTS * EXPERT_DIM), 1) // EXPERT_DIM
    mask = (col_expert == eids).astype(jnp.float32)
    inter = inter_ref[...]
    wide = jnp.concatenate([inter] * NUM_EXPERTS, axis=1) * mask
    out_ref[...] = acc + jnp.dot(wide, wda_ref[...],
                                 preferred_element_type=jnp.float32)


def _final_call(tid, s, inter_un, wsd, wda):
    h = N_EMBD
    nb = N_TOKENS // BN_SH
    return pl.pallas_call(
        _final_body,
        grid=(nb,),
        in_specs=[
            pl.BlockSpec((1, 1, BN_SH), lambda i: (i, 0, 0)),
            pl.BlockSpec((BN_SH, h), lambda i: (i, 0)),
            pl.BlockSpec((BN_SH, EXPERT_DIM), lambda i: (i, 0)),
            pl.BlockSpec((h, h), lambda i: (0, 0)),
            pl.BlockSpec((h, h), lambda i: (0, 0)),
        ],
        out_specs=pl.BlockSpec((BN_SH, h), lambda i: (i, 0)),
        out_shape=jax.ShapeDtypeStruct((N_TOKENS, h), jnp.float32),
    )(tid, s, inter_un, wsd, wda)


def _routed_body(meta_ref, xs_ref, wg_ref, wu_ref, out_ref):
    k = pl.program_id(0)
    lo = meta_ref[2, k]
    hi = meta_ref[3, k]
    first = meta_ref[4, k]
    x = xs_ref[...]
    g = jnp.dot(x, wg_ref[0], preferred_element_type=jnp.float32)
    u = jnp.dot(x, wu_ref[0], preferred_element_type=jnp.float32)
    rows = lax.broadcasted_iota(jnp.int32, (BN_RT, EXPERT_DIM), 0)
    m = ((rows >= lo) & (rows < hi)).astype(jnp.float32)
    r = g * jax.nn.sigmoid(g) * u * m

    @pl.when(first == 1)
    def _():
        out_ref[...] = r

    @pl.when(first == 0)
    def _():
        out_ref[...] = out_ref[...] + r


def _routed_call(meta, xs, gate_proj_w, up_proj_w):
    h = N_EMBD
    grid_spec = pltpu.PrefetchScalarGridSpec(
        num_scalar_prefetch=1,
        grid=(NPAIR,),
        in_specs=[
            pl.BlockSpec((BN_RT, h), lambda k, meta: (meta[0, k], 0)),
            pl.BlockSpec((1, h, EXPERT_DIM), lambda k, meta: (meta[1, k], 0, 0)),
            pl.BlockSpec((1, h, EXPERT_DIM), lambda k, meta: (meta[1, k], 0, 0)),
        ],
        out_specs=pl.BlockSpec((BN_RT, EXPERT_DIM),
                               lambda k, meta: (meta[0, k], 0)),
    )
    return pl.pallas_call(
        _routed_body,
        grid_spec=grid_spec,
        out_shape=jax.ShapeDtypeStruct((N_TOKENS, EXPERT_DIM), jnp.float32),
    )(meta, xs, gate_proj_w, up_proj_w)


def kernel(x, token_ids, gate_proj_w, up_proj_w, down_proj_w, shared_gate_w,
           shared_up_w, shared_down_w):
    b, t, h = x.shape
    flat_x = x.reshape(N_TOKENS, h)
    tids = token_ids.reshape(N_TOKENS).astype(jnp.int32)
    wda = down_proj_w.reshape(NUM_EXPERTS * EXPERT_DIM, h)
    tid3 = token_ids.reshape(N_TOKENS // BN_SH, 1, BN_SH).astype(jnp.int32)

    counts = _make_hist()(tids)
    pos, meta, xs = _make_dispatch()(tids, flat_x, counts)
    s = _sgu_call(flat_x, shared_gate_w, shared_up_w)
    inter_sorted = _routed_call(meta, xs, gate_proj_w, up_proj_w)
    inter_un = _make_unsort()(inter_sorted, pos)
    out = _final_call(tid3, s, inter_un, shared_down_w, wda)
    return out.reshape(b, t, h)


# speedup vs baseline: 1.3300x; 1.1325x over previous
import functools

import jax
import jax.numpy as jnp
from jax import lax
from jax.experimental import pallas as pl
from jax.experimental.pallas import tpu as pltpu
from jax.experimental.pallas import tpu_sc as plsc

NUM_EXPERTS = 8
N_EMBD = 1024
EXPERT_DIM = 128
VOCAB = 100000
N_TOKENS = 4096

NC, NS, L = 2, 16, 16
NW = NC * NS
CHUNK = N_TOKENS // NW
VPW = CHUNK // L
SUB = 32
NSUB = CHUNK // SUB

BN_SH = 1024
BN_RT = 512
_BN_RT_SHIFT = 9
NB_RT = N_TOKENS // BN_RT
NPAIR = NB_RT + NUM_EXPERTS - 1


def _sc_mesh():
    return plsc.VectorSubcoreMesh(core_axis_name="c", subcore_axis_name="s",
                                  num_cores=NC, num_subcores=NS)


def _take(v, idx):
    return lax.gather(
        v, idx[:, None],
        dimension_numbers=lax.GatherDimensionNumbers(
            offset_dims=(), collapsed_slice_dims=(0,), start_index_map=(0,)),
        slice_sizes=(1,),
        mode=lax.GatherScatterMode.PROMISE_IN_BOUNDS)


_LANES = None


def _vhelpers():
    lanes = lax.iota(jnp.int32, L)
    ones = jnp.ones((L,), jnp.int32)
    zeros = jnp.zeros((L,), jnp.int32)
    last = jnp.full((L,), L - 1, jnp.int32)

    def b2i(m):
        return jnp.where(m, ones, zeros)

    def vcumsum(v):
        for s in (1, 2, 4, 8):
            sh = _take(v, jnp.maximum(lanes - s, 0))
            v = v + jnp.where(lanes >= s, sh, zeros)
        return v

    def vbroadcast_last(v):
        return _take(v, last)

    return lanes, ones, zeros, b2i, vcumsum, vbroadcast_last


def _expert_of_vec(v):
    v = jnp.minimum(jnp.maximum(v, 0), VOCAB - 1)
    return lax.rem(v, NUM_EXPERTS)


def _hist_body(tids_hbm, counts_hbm, tids_v, cnt_v):
    wid = lax.axis_index("s") * NC + lax.axis_index("c")
    base = wid * CHUNK
    lanes, ones, zeros, b2i, vcumsum, vlast = _vhelpers()
    pltpu.sync_copy(tids_hbm.at[pl.ds(base, CHUNK)], tids_v)
    cnt = zeros
    for j in range(VPW):
        e = _expert_of_vec(tids_v[pl.ds(j * L, L)])
        for ex in range(NUM_EXPERTS):
            pc = vlast(vcumsum(b2i(e == ex)))
            cnt = cnt + jnp.where(lanes == ex, pc, zeros)
    cnt_v[...] = cnt
    pltpu.sync_copy(cnt_v, counts_hbm.at[wid])


def _make_hist():
    return functools.partial(
        pl.kernel,
        out_type=jax.ShapeDtypeStruct((NW, L), jnp.int32),
        mesh=_sc_mesh(),
        scratch_types=[
            pltpu.VMEM((CHUNK,), jnp.int32),
            pltpu.VMEM((L,), jnp.int32),
        ],
    )(_hist_body)


def _rotate(v, lanes, sh):
    return _take(v, (lanes + sh) & (L - 1))


def _vsort16(v, lanes, zeros, ones):
    rank = zeros
    for sh in range(1, L):
        w = _rotate(v, lanes, sh)
        lt = jnp.where(w < v, ones, zeros)
        tie = jnp.where(w == v, jnp.where(lanes >= L - sh, ones, zeros), zeros)
        rank = rank + lt + tie
    out = zeros
    for sh in range(L):
        w = _rotate(v, lanes, sh) if sh else v
        r = _rotate(rank, lanes, sh) if sh else rank
        out = out + jnp.where(r == lanes, w, zeros)
    return out


def _dispatch_body(tids_hbm, x_hbm, counts_hbm, pos_hbm, meta_hbm, xs_hbm,
                   tids_v, cnts_v, meta_v, idx0, idx1, idx2,
                   idx3, xbuf0, xbuf1, sem):
    wid = lax.axis_index("s") * NC + lax.axis_index("c")
    base = wid * CHUNK
    idxrefs = (idx0, idx1, idx2, idx3)
    lanes, ones, zeros, b2i, vcumsum, vlast = _vhelpers()

    pltpu.sync_copy(tids_hbm.at[pl.ds(base, CHUNK)], tids_v)
    pltpu.sync_copy(counts_hbm, cnts_v)

    tot = zeros
    pre = zeros
    for w in range(NW):
        row = cnts_v[w]
        tot = tot + row
        pre = pre + jnp.where(w < wid, row, zeros)

    seg_vec = vcumsum(tot) - tot

    bp = jnp.where(lanes < NB_RT, lanes * BN_RT,
                   _take(seg_vec, jnp.maximum(lanes - (NB_RT - 1), 0)))
    bp = jnp.where(lanes == L - 1, jnp.full((L,), N_TOKENS, jnp.int32), bp)
    bp = _vsort16(bp, lanes, zeros, ones)
    ends = jnp.where(lanes == L - 1, jnp.full((L,), N_TOKENS, jnp.int32),
                     _rotate(bp, lanes, 1))
    blk = jnp.minimum(
        lax.shift_right_logical(bp, jnp.full((L,), _BN_RT_SHIFT, jnp.int32)),
        jnp.full((L,), NB_RT - 1, jnp.int32))
    e_of = zeros - 1
    for e in range(NUM_EXPERTS):
        sege = _take(seg_vec, jnp.full((L,), e, jnp.int32))
        e_of = e_of + jnp.where(sege <= bp, ones, zeros)
    e_of = jnp.minimum(jnp.maximum(e_of, 0),
                       jnp.full((L,), NUM_EXPERTS - 1, jnp.int32))
    blk_lo = blk * BN_RT
    first = jnp.where(lanes == 0, ones,
                      jnp.where(blk != _rotate(blk, lanes, L - 1),
                                ones, zeros))

    @pl.when(wid == 0)
    def _():
        meta_v[0, pl.ds(0, L)] = blk
        meta_v[1, pl.ds(0, L)] = e_of
        meta_v[2, pl.ds(0, L)] = bp - blk_lo
        meta_v[3, pl.ds(0, L)] = ends - blk_lo
        meta_v[4, pl.ds(0, L)] = first
        pltpu.sync_copy(meta_v, meta_hbm)

    run = seg_vec + pre
    for j in range(VPW):
        e = _expert_of_vec(tids_v[pl.ds(j * L, L)])
        posv = zeros
        for ex in range(NUM_EXPERTS):
            m = e == ex
            c = vcumsum(b2i(m))
            runex = _take(run, jnp.full((L,), ex, jnp.int32))
            posv = jnp.where(m, runex + c - 1, posv)
            run = run + jnp.where(lanes == ex, vlast(c), zeros)
        idxrefs[j // 2][pl.ds((j % 2) * L, L)] = posv

    for k in range(NSUB):
        pltpu.sync_copy(idxrefs[k], pos_hbm.at[pl.ds(base + k * SUB, SUB)])

    xbufs = (xbuf0, xbuf1)
    pltpu.sync_copy(x_hbm.at[pl.ds(base, SUB)], xbuf0)
    cps = [None, None]
    for k in range(NSUB):
        cps[k % 2] = pltpu.async_copy(xbufs[k % 2], xs_hbm.at[idxrefs[k]], sem)
        if k + 1 < NSUB:
            if cps[(k + 1) % 2] is not None:
                cps[(k + 1) % 2].wait()
            pltpu.sync_copy(x_hbm.at[pl.ds(base + (k + 1) * SUB, SUB)],
                            xbufs[(k + 1) % 2])
    cps[0].wait()
    cps[1].wait()


def _make_dispatch():
    return functools.partial(
        pl.kernel,
        out_type=[
            jax.ShapeDtypeStruct((N_TOKENS,), jnp.int32),
            jax.ShapeDtypeStruct((5, L), jnp.int32),
            jax.ShapeDtypeStruct((N_TOKENS, N_EMBD), jnp.float32),
        ],
        mesh=_sc_mesh(),
        scratch_types=[
            pltpu.VMEM((CHUNK,), jnp.int32),
            pltpu.VMEM((NW, L), jnp.int32),
            pltpu.VMEM((5, L), jnp.int32),
            pltpu.VMEM((SUB,), jnp.int32),
            pltpu.VMEM((SUB,), jnp.int32),
            pltpu.VMEM((SUB,), jnp.int32),
            pltpu.VMEM((SUB,), jnp.int32),
            pltpu.VMEM((SUB, N_EMBD), jnp.float32),
            pltpu.VMEM((SUB, N_EMBD), jnp.float32),
            pltpu.SemaphoreType.DMA,
        ],
    )(_dispatch_body)


def _unsort_body(is_hbm, pos_hbm, out_hbm, idxbuf, gbuf, sem):
    wid = lax.axis_index("s") * NC + lax.axis_index("c")
    base = wid * CHUNK
    pltpu.sync_copy(pos_hbm.at[pl.ds(base, CHUNK)], idxbuf)
    pltpu.async_copy(is_hbm.at[idxbuf], gbuf, sem).wait()
    pltpu.sync_copy(gbuf, out_hbm.at[pl.ds(base, CHUNK)])


def _make_unsort():
    return functools.partial(
        pl.kernel,
        out_type=jax.ShapeDtypeStruct((N_TOKENS, EXPERT_DIM), jnp.float32),
        mesh=_sc_mesh(),
        scratch_types=[
            pltpu.VMEM((CHUNK,), jnp.int32),
            pltpu.VMEM((CHUNK, EXPERT_DIM), jnp.float32),
            pltpu.SemaphoreType.DMA,
        ],
    )(_unsort_body)


def _sgu_body(x_ref, wsg_ref, wsu_ref, out_ref):
    x = x_ref[...]
    g = jnp.dot(x, wsg_ref[...], preferred_element_type=jnp.float32)
    u = jnp.dot(x, wsu_ref[...], preferred_element_type=jnp.float32)
    out_ref[...] = g * jax.nn.sigmoid(g) * u


def _sgu_call(flat_x, wsg, wsu):
    h = N_EMBD
    return pl.pallas_call(
        _sgu_body,
        grid=(N_TOKENS // BN_SH,),
        in_specs=[
            pl.BlockSpec((BN_SH, h), lambda i: (i, 0)),
            pl.BlockSpec((h, h), lambda i: (0, 0)),
            pl.BlockSpec((h, h), lambda i: (0, 0)),
        ],
        out_specs=pl.BlockSpec((BN_SH, h), lambda i: (i, 0)),
        out_shape=jax.ShapeDtypeStruct((N_TOKENS, h), jnp.float32),
    )(flat_x, wsg, wsu)


def _final_body(tid_ref, s_ref, inter_ref, wsd_ref, wda_ref, out_ref):
    s = s_ref[...]
    acc = jnp.dot(s, wsd_ref[...], preferred_element_type=jnp.float32)
    eids = lax.rem(tid_ref[0, 0, :], NUM_EXPERTS).reshape(BN_SH, 1)
    col_expert = lax.broadcasted_iota(
        jnp.int32, (BN_SH, NUM_EXPERTS * EXPERT_DIM), 1) // EXPERT_DIM
    mask = (col_expert == eids).astype(jnp.float32)
    inter = inter_ref[...]
    wide = jnp.concatenate([inter] * NUM_EXPERTS, axis=1) * mask
    out_ref[...] = acc + jnp.dot(wide, wda_ref[...],
                                 preferred_element_type=jnp.float32)


def _final_call(tid, s, inter_un, wsd, wda):
    h = N_EMBD
    nb = N_TOKENS // BN_SH
    return pl.pallas_call(
        _final_body,
        grid=(nb,),
        in_specs=[
            pl.BlockSpec((1, 1, BN_SH), lambda i: (i, 0, 0)),
            pl.BlockSpec((BN_SH, h), lambda i: (i, 0)),
            pl.BlockSpec((BN_SH, EXPERT_DIM), lambda i: (i, 0)),
            pl.BlockSpec((h, h), lambda i: (0, 0)),
            pl.BlockSpec((h, h), lambda i: (0, 0)),
        ],
        out_specs=pl.BlockSpec((BN_SH, h), lambda i: (i, 0)),
        out_shape=jax.ShapeDtypeStruct((N_TOKENS, h), jnp.float32),
    )(tid, s, inter_un, wsd, wda)


def _routed_body(meta_ref, xs_ref, wg_ref, wu_ref, out_ref):
    k = pl.program_id(0)
    e = meta_ref[1, k]
    lo = meta_ref[2, k]
    hi = meta_ref[3, k]
    first = meta_ref[4, k]
    x = xs_ref[...]
    wg = wg_ref[pl.ds(pl.multiple_of(e, 1), 1)][0]
    wu = wu_ref[pl.ds(pl.multiple_of(e, 1), 1)][0]
    g = jnp.dot(x, wg, preferred_element_type=jnp.float32)
    u = jnp.dot(x, wu, preferred_element_type=jnp.float32)
    rows = lax.broadcasted_iota(jnp.int32, (BN_RT, EXPERT_DIM), 0)
    m = ((rows >= lo) & (rows < hi)).astype(jnp.float32)
    r = g * jax.nn.sigmoid(g) * u * m

    @pl.when(first == 1)
    def _():
        out_ref[...] = r

    @pl.when(first == 0)
    def _():
        out_ref[...] = out_ref[...] + r


def _routed_call(meta, xs, gate_proj_w, up_proj_w):
    h = N_EMBD
    grid_spec = pltpu.PrefetchScalarGridSpec(
        num_scalar_prefetch=1,
        grid=(NPAIR,),
        in_specs=[
            pl.BlockSpec((BN_RT, h), lambda k, meta: (meta[0, k], 0)),
            pl.BlockSpec((NUM_EXPERTS, h, EXPERT_DIM), lambda k, meta: (0, 0, 0)),
            pl.BlockSpec((NUM_EXPERTS, h, EXPERT_DIM), lambda k, meta: (0, 0, 0)),
        ],
        out_specs=pl.BlockSpec((BN_RT, EXPERT_DIM),
                               lambda k, meta: (meta[0, k], 0)),
    )
    return pl.pallas_call(
        _routed_body,
        grid_spec=grid_spec,
        out_shape=jax.ShapeDtypeStruct((N_TOKENS, EXPERT_DIM), jnp.float32),
    )(meta, xs, gate_proj_w, up_proj_w)


def kernel(x, token_ids, gate_proj_w, up_proj_w, down_proj_w, shared_gate_w,
           shared_up_w, shared_down_w):
    b, t, h = x.shape
    flat_x = x.reshape(N_TOKENS, h)
    tids = token_ids.reshape(N_TOKENS).astype(jnp.int32)
    wda = down_proj_w.reshape(NUM_EXPERTS * EXPERT_DIM, h)
    tid3 = token_ids.reshape(N_TOKENS // BN_SH, 1, BN_SH).astype(jnp.int32)

    counts = _make_hist()(tids)
    pos, meta, xs = _make_dispatch()(tids, flat_x, counts)
    s = _sgu_call(flat_x, shared_gate_w, shared_up_w)
    xs_b, s = lax.optimization_barrier((xs, s))
    inter_sorted = _routed_call(meta, xs_b, gate_proj_w, up_proj_w)
    inter_un = _make_unsort()(inter_sorted, pos)
    out = _final_call(tid3, s, inter_un, shared_down_w, wda)
    return out.reshape(b, t, h)


# speedup vs baseline: 1.3323x; 1.0018x over previous
import functools

import jax
import jax.numpy as jnp
from jax import lax
from jax.experimental import pallas as pl
from jax.experimental.pallas import tpu as pltpu
from jax.experimental.pallas import tpu_sc as plsc

NUM_EXPERTS = 8
N_EMBD = 1024
EXPERT_DIM = 128
VOCAB = 100000
N_TOKENS = 4096

NC, NS, L = 2, 16, 16
NW = NC * NS
CHUNK = N_TOKENS // NW
VPW = CHUNK // L
SUB = 32
NSUB = CHUNK // SUB

BN_SH = 1024
BN_RT = 512
_BN_RT_SHIFT = 9
NB_RT = N_TOKENS // BN_RT
NPAIR = NB_RT + NUM_EXPERTS - 1


def _sc_mesh():
    return plsc.VectorSubcoreMesh(core_axis_name="c", subcore_axis_name="s",
                                  num_cores=NC, num_subcores=NS)


def _take(v, idx):
    return lax.gather(
        v, idx[:, None],
        dimension_numbers=lax.GatherDimensionNumbers(
            offset_dims=(), collapsed_slice_dims=(0,), start_index_map=(0,)),
        slice_sizes=(1,),
        mode=lax.GatherScatterMode.PROMISE_IN_BOUNDS)


_LANES = None


def _vhelpers():
    lanes = lax.iota(jnp.int32, L)
    ones = jnp.ones((L,), jnp.int32)
    zeros = jnp.zeros((L,), jnp.int32)
    last = jnp.full((L,), L - 1, jnp.int32)

    def b2i(m):
        return jnp.where(m, ones, zeros)

    def vcumsum(v):
        for s in (1, 2, 4, 8):
            sh = _take(v, jnp.maximum(lanes - s, 0))
            v = v + jnp.where(lanes >= s, sh, zeros)
        return v

    def vbroadcast_last(v):
        return _take(v, last)

    return lanes, ones, zeros, b2i, vcumsum, vbroadcast_last


def _expert_of_vec(v):
    v = jnp.minimum(jnp.maximum(v, 0), VOCAB - 1)
    return lax.rem(v, NUM_EXPERTS)


def _hist_body(tids_hbm, counts_hbm, tids_v, cnt_v):
    wid = lax.axis_index("s") * NC + lax.axis_index("c")
    base = wid * CHUNK
    lanes, ones, zeros, b2i, vcumsum, vlast = _vhelpers()
    pltpu.sync_copy(tids_hbm.at[pl.ds(base, CHUNK)], tids_v)
    cnt = zeros
    for j in range(VPW):
        e = _expert_of_vec(tids_v[pl.ds(j * L, L)])
        for ex in range(NUM_EXPERTS):
            pc = vlast(vcumsum(b2i(e == ex)))
            cnt = cnt + jnp.where(lanes == ex, pc, zeros)
    cnt_v[...] = cnt
    pltpu.sync_copy(cnt_v, counts_hbm.at[wid])


def _make_hist():
    return functools.partial(
        pl.kernel,
        out_type=jax.ShapeDtypeStruct((NW, L), jnp.int32),
        mesh=_sc_mesh(),
        scratch_types=[
            pltpu.VMEM((CHUNK,), jnp.int32),
            pltpu.VMEM((L,), jnp.int32),
        ],
    )(_hist_body)


def _rotate(v, lanes, sh):
    return _take(v, (lanes + sh) & (L - 1))


def _vsort16(v, lanes, zeros, ones):
    rank = zeros
    for sh in range(1, L):
        w = _rotate(v, lanes, sh)
        lt = jnp.where(w < v, ones, zeros)
        tie = jnp.where(w == v, jnp.where(lanes >= L - sh, ones, zeros), zeros)
        rank = rank + lt + tie
    out = zeros
    for sh in range(L):
        w = _rotate(v, lanes, sh) if sh else v
        r = _rotate(rank, lanes, sh) if sh else rank
        out = out + jnp.where(r == lanes, w, zeros)
    return out


def _dispatch_body(tids_hbm, x_hbm, counts_hbm, pos_hbm, meta_hbm, xs_hbm,
                   tids_v, cnts_v, meta_v, idx0, idx1, idx2,
                   idx3, xbuf0, xbuf1, sem):
    wid = lax.axis_index("s") * NC + lax.axis_index("c")
    base = wid * CHUNK
    idxrefs = (idx0, idx1, idx2, idx3)
    lanes, ones, zeros, b2i, vcumsum, vlast = _vhelpers()

    pltpu.sync_copy(tids_hbm.at[pl.ds(base, CHUNK)], tids_v)
    pltpu.sync_copy(counts_hbm, cnts_v)

    tot = zeros
    pre = zeros
    for w in range(NW):
        row = cnts_v[w]
        tot = tot + row
        pre = pre + jnp.where(w < wid, row, zeros)

    seg_vec = vcumsum(tot) - tot

    bp = jnp.where(lanes < NB_RT, lanes * BN_RT,
                   _take(seg_vec, jnp.maximum(lanes - (NB_RT - 1), 0)))
    bp = jnp.where(lanes == L - 1, jnp.full((L,), N_TOKENS, jnp.int32), bp)
    bp = _vsort16(bp, lanes, zeros, ones)
    ends = jnp.where(lanes == L - 1, jnp.full((L,), N_TOKENS, jnp.int32),
                     _rotate(bp, lanes, 1))
    blk = jnp.minimum(
        lax.shift_right_logical(bp, jnp.full((L,), _BN_RT_SHIFT, jnp.int32)),
        jnp.full((L,), NB_RT - 1, jnp.int32))
    e_of = zeros - 1
    for e in range(NUM_EXPERTS):
        sege = _take(seg_vec, jnp.full((L,), e, jnp.int32))
        e_of = e_of + jnp.where(sege <= bp, ones, zeros)
    e_of = jnp.minimum(jnp.maximum(e_of, 0),
                       jnp.full((L,), NUM_EXPERTS - 1, jnp.int32))
    blk_lo = blk * BN_RT
    first = jnp.where(lanes == 0, ones,
                      jnp.where(blk != _rotate(blk, lanes, L - 1),
                                ones, zeros))

    @pl.when(wid == 0)
    def _():
        meta_v[0, pl.ds(0, L)] = blk
        meta_v[1, pl.ds(0, L)] = e_of
        meta_v[2, pl.ds(0, L)] = bp - blk_lo
        meta_v[3, pl.ds(0, L)] = ends - blk_lo
        meta_v[4, pl.ds(0, L)] = first
        pltpu.sync_copy(meta_v, meta_hbm)

    run = seg_vec + pre
    for j in range(VPW):
        e = _expert_of_vec(tids_v[pl.ds(j * L, L)])
        posv = zeros
        for ex in range(NUM_EXPERTS):
            m = e == ex
            c = vcumsum(b2i(m))
            runex = _take(run, jnp.full((L,), ex, jnp.int32))
            posv = jnp.where(m, runex + c - 1, posv)
            run = run + jnp.where(lanes == ex, vlast(c), zeros)
        idxrefs[j // 2][pl.ds((j % 2) * L, L)] = posv

    for k in range(NSUB):
        pltpu.sync_copy(idxrefs[k], pos_hbm.at[pl.ds(base + k * SUB, SUB)])

    xbufs = (xbuf0, xbuf1)
    pltpu.sync_copy(x_hbm.at[pl.ds(base, SUB)], xbuf0)
    cps = [None, None]
    for k in range(NSUB):
        cps[k % 2] = pltpu.async_copy(xbufs[k % 2], xs_hbm.at[idxrefs[k]], sem)
        if k + 1 < NSUB:
            if cps[(k + 1) % 2] is not None:
                cps[(k + 1) % 2].wait()
            pltpu.sync_copy(x_hbm.at[pl.ds(base + (k + 1) * SUB, SUB)],
                            xbufs[(k + 1) % 2])
    cps[0].wait()
    cps[1].wait()


def _make_dispatch():
    return functools.partial(
        pl.kernel,
        out_type=[
            jax.ShapeDtypeStruct((N_TOKENS,), jnp.int32),
            jax.ShapeDtypeStruct((5, L), jnp.int32),
            jax.ShapeDtypeStruct((N_TOKENS, N_EMBD), jnp.float32),
        ],
        mesh=_sc_mesh(),
        scratch_types=[
            pltpu.VMEM((CHUNK,), jnp.int32),
            pltpu.VMEM((NW, L), jnp.int32),
            pltpu.VMEM((5, L), jnp.int32),
            pltpu.VMEM((SUB,), jnp.int32),
            pltpu.VMEM((SUB,), jnp.int32),
            pltpu.VMEM((SUB,), jnp.int32),
            pltpu.VMEM((SUB,), jnp.int32),
            pltpu.VMEM((SUB, N_EMBD), jnp.float32),
            pltpu.VMEM((SUB, N_EMBD), jnp.float32),
            pltpu.SemaphoreType.DMA,
        ],
    )(_dispatch_body)


def _unsort_body(is_hbm, pos_hbm, out_hbm, idxbuf, gbuf, sem):
    wid = lax.axis_index("s") * NC + lax.axis_index("c")
    base = wid * CHUNK
    pltpu.sync_copy(pos_hbm.at[pl.ds(base, CHUNK)], idxbuf)
    pltpu.async_copy(is_hbm.at[idxbuf], gbuf, sem).wait()
    pltpu.sync_copy(gbuf, out_hbm.at[pl.ds(base, CHUNK)])


def _make_unsort():
    return functools.partial(
        pl.kernel,
        out_type=jax.ShapeDtypeStruct((N_TOKENS, EXPERT_DIM), jnp.float32),
        mesh=_sc_mesh(),
        scratch_types=[
            pltpu.VMEM((CHUNK,), jnp.int32),
            pltpu.VMEM((CHUNK, EXPERT_DIM), jnp.float32),
            pltpu.SemaphoreType.DMA,
        ],
    )(_unsort_body)


def _sgu_body(x_ref, wsg_ref, wsu_ref, out_ref):
    x = x_ref[...]
    g = jnp.dot(x, wsg_ref[...], preferred_element_type=jnp.float32)
    u = jnp.dot(x, wsu_ref[...], preferred_element_type=jnp.float32)
    out_ref[...] = g * jax.nn.sigmoid(g) * u


def _sgu_call(flat_x, wsg, wsu):
    h = N_EMBD
    return pl.pallas_call(
        _sgu_body,
        grid=(N_TOKENS // BN_SH,),
        in_specs=[
            pl.BlockSpec((BN_SH, h), lambda i: (i, 0)),
            pl.BlockSpec((h, h), lambda i: (0, 0)),
            pl.BlockSpec((h, h), lambda i: (0, 0)),
        ],
        out_specs=pl.BlockSpec((BN_SH, h), lambda i: (i, 0)),
        out_shape=jax.ShapeDtypeStruct((N_TOKENS, h), jnp.float32),
    )(flat_x, wsg, wsu)


def _final_body(tid_ref, s_ref, inter_ref, wsd_ref, wda_ref, out_ref):
    s = s_ref[...]
    acc = jnp.dot(s, wsd_ref[...], preferred_element_type=jnp.float32)
    eids = lax.rem(tid_ref[0, 0, :], NUM_EXPERTS).reshape(BN_SH, 1)
    col_expert = lax.broadcasted_iota(
        jnp.int32, (BN_SH, NUM_EXPERTS * EXPERT_DIM), 1) // EXPERT_DIM
    mask = (col_expert == eids).astype(jnp.float32)
    inter = inter_ref[...]
    wide = jnp.concatenate([inter] * NUM_EXPERTS, axis=1) * mask
    out_ref[...] = acc + jnp.dot(wide, wda_ref[...],
                                 preferred_element_type=jnp.float32)


def _final_call(tid, s, inter_un, wsd, wda):
    h = N_EMBD
    nb = N_TOKENS // BN_SH
    return pl.pallas_call(
        _final_body,
        grid=(nb,),
        in_specs=[
            pl.BlockSpec((1, 1, BN_SH), lambda i: (i, 0, 0)),
            pl.BlockSpec((BN_SH, h), lambda i: (i, 0)),
            pl.BlockSpec((BN_SH, EXPERT_DIM), lambda i: (i, 0)),
            pl.BlockSpec((h, h), lambda i: (0, 0)),
            pl.BlockSpec((h, h), lambda i: (0, 0)),
        ],
        out_specs=pl.BlockSpec((BN_SH, h), lambda i: (i, 0)),
        out_shape=jax.ShapeDtypeStruct((N_TOKENS, h), jnp.float32),
    )(tid, s, inter_un, wsd, wda)


def _routed_body(meta_ref, xs_ref, wg_ref, wu_ref, out_ref):
    k = pl.program_id(0)
    e = meta_ref[1, k]
    lo = meta_ref[2, k]
    hi = meta_ref[3, k]
    first = meta_ref[4, k]
    x = xs_ref[...]
    off = pl.multiple_of(e * N_EMBD, N_EMBD)
    wg = wg_ref[pl.ds(off, N_EMBD), :]
    wu = wu_ref[pl.ds(off, N_EMBD), :]
    g = jnp.dot(x, wg, preferred_element_type=jnp.float32)
    u = jnp.dot(x, wu, preferred_element_type=jnp.float32)
    rows = lax.broadcasted_iota(jnp.int32, (BN_RT, EXPERT_DIM), 0)
    m = ((rows >= lo) & (rows < hi)).astype(jnp.float32)
    r = g * jax.nn.sigmoid(g) * u * m

    @pl.when(first == 1)
    def _():
        out_ref[...] = r

    @pl.when(first == 0)
    def _():
        out_ref[...] = out_ref[...] + r


def _routed_call(meta, xs, gate_proj_w, up_proj_w):
    h = N_EMBD
    grid_spec = pltpu.PrefetchScalarGridSpec(
        num_scalar_prefetch=1,
        grid=(NPAIR,),
        in_specs=[
            pl.BlockSpec((BN_RT, h), lambda k, meta: (meta[0, k], 0)),
            pl.BlockSpec((NUM_EXPERTS * h, EXPERT_DIM), lambda k, meta: (0, 0)),
            pl.BlockSpec((NUM_EXPERTS * h, EXPERT_DIM), lambda k, meta: (0, 0)),
        ],
        out_specs=pl.BlockSpec((BN_RT, EXPERT_DIM),
                               lambda k, meta: (meta[0, k], 0)),
    )
    return pl.pallas_call(
        _routed_body,
        grid_spec=grid_spec,
        out_shape=jax.ShapeDtypeStruct((N_TOKENS, EXPERT_DIM), jnp.float32),
    )(meta, xs,
      gate_proj_w.reshape(NUM_EXPERTS * h, EXPERT_DIM),
      up_proj_w.reshape(NUM_EXPERTS * h, EXPERT_DIM))


def kernel(x, token_ids, gate_proj_w, up_proj_w, down_proj_w, shared_gate_w,
           shared_up_w, shared_down_w):
    b, t, h = x.shape
    flat_x = x.reshape(N_TOKENS, h)
    tids = token_ids.reshape(N_TOKENS).astype(jnp.int32)
    wda = down_proj_w.reshape(NUM_EXPERTS * EXPERT_DIM, h)
    tid3 = token_ids.reshape(N_TOKENS // BN_SH, 1, BN_SH).astype(jnp.int32)

    counts = _make_hist()(tids)
    pos, meta, xs = _make_dispatch()(tids, flat_x, counts)
    s = _sgu_call(flat_x, shared_gate_w, shared_up_w)
    xs_b, s = lax.optimization_barrier((xs, s))
    inter_sorted = _routed_call(meta, xs_b, gate_proj_w, up_proj_w)
    inter_un = _make_unsort()(inter_sorted, pos)
    out = _final_call(tid3, s, inter_un, shared_down_w, wda)
    return out.reshape(b, t, h)
